# Initial kernel scaffold; baseline (speedup 1.0000x reference)
import jax
import jax.numpy as jnp
from jax.experimental import pallas as pl


def kernel(topic_ids, table, W1, b1, W2, b2):
    raise NotImplementedError("write your pallas kernel here")


# speedup vs baseline: 13.1488x; 13.1488x over previous
import functools

import jax
import jax.numpy as jnp
from jax import lax
from jax.experimental import pallas as pl
from jax.experimental.pallas import tpu as pltpu
from jax.experimental.pallas import tpu_sc as plsc

_NT = 18115
_D = 64
_B = 16384
_L = 50
_TP = 18176

_NW = 32
_IPW = _B // _NW
_IPB = 64
_CPB = _IPB // 2
_NBLK = _IPW // _IPB


def _scores_tc(table_p, w1t, b1, w2r, b2):
    def body(tp_ref, w1t_ref, b1_ref, w2_ref, b2_ref, s_ref):
        e = jnp.tanh(
            jnp.dot(tp_ref[...], w1t_ref[...],
                    preferred_element_type=jnp.float32)
            + b1_ref[...][None, :]
        )
        z = jnp.sum(e * w2_ref[...][None, :], axis=1) + b2_ref[0]
        s_ref[...] = jnp.exp(z)

    return pl.pallas_call(
        body,
        out_shape=jax.ShapeDtypeStruct((_TP,), jnp.float32),
    )(table_p, w1t, b1, w2r, b2)


def _make_sc_bag():
    mesh = plsc.VectorSubcoreMesh(core_axis_name="c", subcore_axis_name="s")

    @functools.partial(
        pl.kernel,
        mesh=mesh,
        compiler_params=pltpu.CompilerParams(
            needs_layout_passes=False, use_tc_tiling_on_sc=False),
        out_type=jax.ShapeDtypeStruct((_B, _D), jnp.float32),
        scratch_types=[
            pltpu.VMEM((_TP,), jnp.float32),
            pltpu.VMEM((_CPB, 100), jnp.int32),
            pltpu.VMEM((_CPB, 128), jnp.int32),
            pltpu.VMEM((100, _D), jnp.float32),
            pltpu.VMEM((_IPB, _D), jnp.float32),
            pltpu.SemaphoreType.DMA,
        ],
    )
    def sc_bag(ids2d_hbm, ids2dp_hbm, table_hbm, s_hbm, out_hbm,
               s_v, ids_vg, ids_vw, rows_v, out_v, sem):
        wid = lax.axis_index("s") * 2 + lax.axis_index("c")
        pltpu.sync_copy(s_hbm, s_v)
        item0 = wid * _IPW

        def block_body(blk, carry):
            ib = pl.multiple_of(item0 + blk * _IPB, _IPB)
            crow = pl.multiple_of(ib // 2, _IPB // 2)
            pltpu.sync_copy(ids2d_hbm.at[pl.ds(crow, _CPB)], ids_vg)
            pltpu.sync_copy(ids2dp_hbm.at[pl.ds(crow, _CPB)], ids_vw)

            def chunk_body(j, c2):
                pltpu.async_copy(table_hbm.at[ids_vg.at[j]], rows_v, sem).wait()
                for a in range(2):
                    idv = [ids_vw[j, pl.ds(a * _L + k * 16, 16)]
                           for k in range(4)]
                    wv = [plsc.load_gather(s_v, [idv[k]]) for k in range(4)]
                    zero = jnp.zeros((16,), jnp.float32)
                    wv = [jnp.where(idv[k] == 0, zero, wv[k])
                          for k in range(4)]
                    lanes = lax.iota(jnp.int32, 16)
                    wv[3] = jnp.where(lanes < (_L - 48), wv[3], zero)
                    den = jnp.sum(wv[0] + wv[1] + wv[2] + wv[3])
                    dv = jnp.full((16,), den + jnp.float32(1e-8), jnp.float32)
                    wn = [wv[k] / dv for k in range(4)]
                    acc = [jnp.zeros((16,), jnp.float32) for _ in range(4)]
                    for l in range(_L):
                        t = a * _L + l
                        wvb = jnp.full((16,), wn[l // 16][l % 16], jnp.float32)
                        for dd in range(4):
                            acc[dd] = acc[dd] + wvb * rows_v[t, pl.ds(dd * 16, 16)]
                    row = 2 * j + a
                    for dd in range(4):
                        out_v[row, pl.ds(dd * 16, 16)] = acc[dd]
                return c2

            lax.fori_loop(0, _CPB, chunk_body, 0)
            pltpu.sync_copy(out_v, out_hbm.at[pl.ds(ib, _IPB)])
            return carry

        lax.fori_loop(0, _NBLK, block_body, 0)

    return sc_bag


_sc_bag = _make_sc_bag()


def kernel(topic_ids, table, W1, b1, W2, b2):
    ids2d = topic_ids.astype(jnp.int32).reshape(_B * _L // 100, 100)
    ids2dp = jnp.pad(ids2d, ((0, 0), (0, 28)))
    table_p = jnp.pad(table, ((0, _TP - _NT), (0, 0)))
    s = _scores_tc(table_p, W1.T, b1, W2[0], b2)
    return _sc_bag(ids2d, ids2dp, table, s)
